# baseline (speedup 1.0000x reference)
import functools

import jax
import jax.numpy as jnp
from jax import lax
from jax.experimental import pallas as pl
from jax.experimental.pallas import tpu as pltpu
from jax.experimental.pallas import tpu_sc as plsc

N, K, C = 10000, 32, 128
NK = N * K

_NC, _NS = 2, 16
_NW = _NC * _NS
_NBUF = 5

_BLK = 400

_SLABS = ((0, 3200), (3200, 3200), (6400, 3600))

_INV_SQRT2 = 0.7071067811865476


def _gelu_exact(x):
    return 0.5 * x * (1.0 + lax.erf(x * _INV_SQRT2))


def _pre_body(feat_ref, w1_ref, b1_ref, a_ref, bv_ref):
    f = feat_ref[...]
    w1a = w1_ref[:C, :]
    w1b = w1_ref[C:, :]
    a_ref[...] = jnp.dot(f, w1b, preferred_element_type=jnp.float32,
                         precision=lax.Precision.HIGHEST)
    bv_ref[...] = jnp.dot(f, w1a - w1b, preferred_element_type=jnp.float32,
                          precision=lax.Precision.HIGHEST) + b1_ref[...]


def _pre_pass(feat, W1, b1):
    return pl.pallas_call(
        _pre_body,
        grid=(N // _BLK,),
        in_specs=[
            pl.BlockSpec((_BLK, C), lambda i: (i, 0)),
            pl.BlockSpec((2 * C, C), lambda i: (0, 0)),
            pl.BlockSpec((1, C), lambda i: (0, 0)),
        ],
        out_specs=[
            pl.BlockSpec((_BLK, C), lambda i: (i, 0)),
            pl.BlockSpec((_BLK, C), lambda i: (i, 0)),
        ],
        out_shape=[
            jax.ShapeDtypeStruct((N, C), jnp.float32),
            jax.ShapeDtypeStruct((N, C), jnp.float32),
        ],
    )(feat, W1, b1.reshape(1, C))


def _sc_gather_body(nch, ch, a_hbm, idx_hbm, g_hbm, idx_v, r0, r1, r2, r3, r4,
                    si0, si1, si2, si3, si4, so0, so1, so2, so3, so4):
    rows = (r0, r1, r2, r3, r4)
    sin = (si0, si1, si2, si3, si4)
    sout = (so0, so1, so2, so3, so4)
    epw = nch * ch
    wid = lax.axis_index("s") * _NC + lax.axis_index("c")
    pltpu.sync_copy(idx_hbm.at[wid], idx_v)
    base = wid * epw

    pltpu.async_copy(a_hbm.at[idx_v.at[0]], rows[0], sin[0])
    pltpu.async_copy(a_hbm.at[idx_v.at[1]], rows[1], sin[1])

    def step(j, carry):
        c0 = j * _NBUF
        for b in range(_NBUF):
            c = c0 + b
            pltpu.make_async_copy(a_hbm.at[idx_v.at[c]], rows[b],
                                  sin[b]).wait()
            pltpu.async_copy(rows[b], g_hbm.at[pl.ds(base + c * ch, ch)],
                             sout[b])
            nb = (b + 2) % _NBUF
            nc = c + 2

            @pl.when(nc >= _NBUF)
            def _():
                pltpu.make_async_copy(
                    rows[nb], g_hbm.at[pl.ds(base + (c - 3) * ch, ch)],
                    sout[nb]).wait()

            @pl.when(nc < nch)
            def _():
                pltpu.async_copy(a_hbm.at[idx_v.at[nc]], rows[nb], sin[nb])
        return carry

    lax.fori_loop(0, nch // _NBUF, step, 0)
    for c in range(nch - (_NBUF - 2), nch):
        b = c % _NBUF
        pltpu.make_async_copy(rows[b], g_hbm.at[pl.ds(base + c * ch, ch)],
                              sout[b]).wait()


@functools.cache
def _sc_gather_call(nch, ch):
    mesh = plsc.VectorSubcoreMesh(core_axis_name="c", subcore_axis_name="s")
    return pl.kernel(
        functools.partial(_sc_gather_body, nch, ch),
        out_type=jax.ShapeDtypeStruct((_NW * nch * ch, C), jnp.float32),
        mesh=mesh,
        scratch_types=(
            [pltpu.VMEM((nch, ch), jnp.int32)]
            + [pltpu.VMEM((ch, C), jnp.float32)] * _NBUF
            + [pltpu.SemaphoreType.DMA] * (2 * _NBUF)),
    )


def _sc_gather(A, idx_slab, n_nodes):
    ch = 128 if (n_nodes % 128 == 0 and (n_nodes // 128) % _NBUF == 0) else 80
    nch = n_nodes // ch
    assert n_nodes % ch == 0 and nch % _NBUF == 0
    idx = idx_slab.reshape(_NW, nch, ch)
    return _sc_gather_call(nch, ch)(A, idx)


def _main_body(g_ref, bv_ref, feat_ref, w2_ref, b2_ref, gamma_ref, beta_ref,
               o_ref):
    bv = bv_ref[...]
    w2 = w2_ref[...]
    acc = None
    for k in range(K):
        h = _gelu_exact(bv + g_ref[:, k, :])
        hk = jnp.dot(h, w2, preferred_element_type=jnp.float32,
                     precision=lax.Precision.DEFAULT)
        acc = hk if acc is None else jnp.maximum(acc, hk)
    x = acc + b2_ref[...] + feat_ref[...]
    mean = jnp.mean(x, axis=1, keepdims=True)
    var = jnp.mean((x - mean) ** 2, axis=1, keepdims=True)
    o_ref[...] = ((x - mean) * lax.rsqrt(var + 1e-5)) * gamma_ref[...] \
        + beta_ref[...]


def _main_pass(G, Bv, feat, W2, b2, gamma, beta, n0, nn):
    blk0 = n0 // _BLK
    return pl.pallas_call(
        _main_body,
        grid=(nn // _BLK,),
        in_specs=[
            pl.BlockSpec((_BLK, K, C), lambda i: (i, 0, 0)),
            pl.BlockSpec((_BLK, C), lambda i: (i + blk0, 0)),
            pl.BlockSpec((_BLK, C), lambda i: (i + blk0, 0)),
            pl.BlockSpec((C, C), lambda i: (0, 0)),
            pl.BlockSpec((1, C), lambda i: (0, 0)),
            pl.BlockSpec((1, C), lambda i: (0, 0)),
            pl.BlockSpec((1, C), lambda i: (0, 0)),
        ],
        out_specs=pl.BlockSpec((_BLK, C), lambda i: (i, 0)),
        out_shape=jax.ShapeDtypeStruct((nn, C), jnp.float32),
    )(G, Bv, feat, W2, b2.reshape(1, C), gamma.reshape(1, C),
      beta.reshape(1, C))


def kernel(feat, knn_idx, W1, b1, W2, b2, gamma, beta):
    idx = knn_idx.astype(jnp.int32).reshape(N, K)
    A, Bv = _pre_pass(feat, W1, b1)
    outs = []
    for n0, nn in _SLABS:
        G = _sc_gather(A, idx[n0:n0 + nn].reshape(-1), nn)
        outs.append(_main_pass(G.reshape(nn, K, C), Bv, feat, W2, b2,
                               gamma, beta, n0, nn))
    return jnp.concatenate(outs, axis=0)


# speedup vs baseline: 1.1376x; 1.1376x over previous
import functools

import jax
import jax.numpy as jnp
from jax import lax
from jax.experimental import pallas as pl
from jax.experimental.pallas import tpu as pltpu
from jax.experimental.pallas import tpu_sc as plsc

N, K, C = 10000, 32, 128
NK = N * K

_NC, _NS = 2, 16
_NW = _NC * _NS
_NBUF = 5

_BLK = 400

_SLABS = ((0, 3200), (3200, 3200), (6400, 3600))

_INV_SQRT2 = 0.7071067811865476


def _gelu_exact(x):
    return 0.5 * x * (1.0 + lax.erf(x * _INV_SQRT2))


def _pre_body(feat_ref, w1_ref, b1_ref, a_ref, bv_ref):
    f = feat_ref[...]
    w1a = w1_ref[:C, :]
    w1b = w1_ref[C:, :]
    a_ref[...] = jnp.dot(f, w1b, preferred_element_type=jnp.float32,
                         precision=lax.Precision.HIGHEST)
    bv_ref[...] = jnp.dot(f, w1a - w1b, preferred_element_type=jnp.float32,
                          precision=lax.Precision.HIGHEST) + b1_ref[...]


def _pre_pass(feat, W1, b1):
    return pl.pallas_call(
        _pre_body,
        grid=(N // _BLK,),
        in_specs=[
            pl.BlockSpec((_BLK, C), lambda i: (i, 0)),
            pl.BlockSpec((2 * C, C), lambda i: (0, 0)),
            pl.BlockSpec((1, C), lambda i: (0, 0)),
        ],
        out_specs=[
            pl.BlockSpec((_BLK, C), lambda i: (i, 0)),
            pl.BlockSpec((_BLK, C), lambda i: (i, 0)),
        ],
        out_shape=[
            jax.ShapeDtypeStruct((N, C), jnp.float32),
            jax.ShapeDtypeStruct((N, C), jnp.float32),
        ],
    )(feat, W1, b1.reshape(1, C))


def _sc_gather_body(nch, ch, a_hbm, idx_hbm, g_hbm, idx_v, r0, r1, r2, r3, r4,
                    si0, si1, si2, si3, si4, so0, so1, so2, so3, so4):
    rows = (r0, r1, r2, r3, r4)
    sin = (si0, si1, si2, si3, si4)
    sout = (so0, so1, so2, so3, so4)
    epw = nch * ch
    wid = lax.axis_index("s") * _NC + lax.axis_index("c")
    pltpu.sync_copy(idx_hbm.at[wid], idx_v)
    base = wid * epw

    pltpu.async_copy(a_hbm.at[idx_v.at[0]], rows[0], sin[0])
    pltpu.async_copy(a_hbm.at[idx_v.at[1]], rows[1], sin[1])

    def step(j, carry):
        c0 = j * _NBUF
        for b in range(_NBUF):
            c = c0 + b
            pltpu.make_async_copy(a_hbm.at[idx_v.at[c]], rows[b],
                                  sin[b]).wait()
            pltpu.async_copy(rows[b], g_hbm.at[pl.ds(base + c * ch, ch)],
                             sout[b])
            nb = (b + 2) % _NBUF
            nc = c + 2

            @pl.when(nc >= _NBUF)
            def _():
                pltpu.make_async_copy(
                    rows[nb], g_hbm.at[pl.ds(base + (c - 3) * ch, ch)],
                    sout[nb]).wait()

            @pl.when(nc < nch)
            def _():
                pltpu.async_copy(a_hbm.at[idx_v.at[nc]], rows[nb], sin[nb])
        return carry

    lax.fori_loop(0, nch // _NBUF, step, 0)
    for c in range(nch - (_NBUF - 2), nch):
        b = c % _NBUF
        pltpu.make_async_copy(rows[b], g_hbm.at[pl.ds(base + c * ch, ch)],
                              sout[b]).wait()


@functools.cache
def _sc_gather_call(nch, ch):
    mesh = plsc.VectorSubcoreMesh(core_axis_name="c", subcore_axis_name="s")
    return pl.kernel(
        functools.partial(_sc_gather_body, nch, ch),
        out_type=jax.ShapeDtypeStruct((_NW * nch * ch, C), jnp.float32),
        mesh=mesh,
        scratch_types=(
            [pltpu.VMEM((nch, ch), jnp.int32)]
            + [pltpu.VMEM((ch, C), jnp.float32)] * _NBUF
            + [pltpu.SemaphoreType.DMA] * (2 * _NBUF)),
    )


def _sc_gather(A, idx_slab, n_nodes):
    ch = 80
    nch = n_nodes // ch
    assert n_nodes % ch == 0 and nch % _NBUF == 0
    idx = idx_slab.reshape(_NW, nch, ch)
    return _sc_gather_call(nch, ch)(A, idx)


def _main_body(g_ref, bv_ref, feat_ref, w2_ref, b2_ref, gamma_ref, beta_ref,
               o_ref):
    bv = bv_ref[...]
    w2 = w2_ref[...]
    acc = None
    for k in range(K):
        h = _gelu_exact(bv + g_ref[k])
        hk = jnp.dot(h, w2, preferred_element_type=jnp.float32,
                     precision=lax.Precision.DEFAULT)
        acc = hk if acc is None else jnp.maximum(acc, hk)
    x = acc + b2_ref[...] + feat_ref[...]
    mean = jnp.mean(x, axis=1, keepdims=True)
    var = jnp.mean((x - mean) ** 2, axis=1, keepdims=True)
    o_ref[...] = ((x - mean) * lax.rsqrt(var + 1e-5)) * gamma_ref[...] \
        + beta_ref[...]


def _main_pass(G, Bv, feat, W2, b2, gamma, beta, n0, nn):
    blk0 = n0 // _BLK
    return pl.pallas_call(
        _main_body,
        grid=(nn // _BLK,),
        in_specs=[
            pl.BlockSpec((K, _BLK, C), lambda i: (0, i, 0)),
            pl.BlockSpec((_BLK, C), lambda i: (i + blk0, 0)),
            pl.BlockSpec((_BLK, C), lambda i: (i + blk0, 0)),
            pl.BlockSpec((C, C), lambda i: (0, 0)),
            pl.BlockSpec((1, C), lambda i: (0, 0)),
            pl.BlockSpec((1, C), lambda i: (0, 0)),
            pl.BlockSpec((1, C), lambda i: (0, 0)),
        ],
        out_specs=pl.BlockSpec((_BLK, C), lambda i: (i, 0)),
        out_shape=jax.ShapeDtypeStruct((nn, C), jnp.float32),
    )(G, Bv, feat, W2, b2.reshape(1, C), gamma.reshape(1, C),
      beta.reshape(1, C))


def kernel(feat, knn_idx, W1, b1, W2, b2, gamma, beta):
    idx_t = knn_idx.astype(jnp.int32).reshape(N, K).T
    A, Bv = _pre_pass(feat, W1, b1)
    outs = []
    for n0, nn in _SLABS:
        G = _sc_gather(A, idx_t[:, n0:n0 + nn].reshape(-1), nn)
        outs.append(_main_pass(G.reshape(K, nn, C), Bv, feat, W2, b2,
                               gamma, beta, n0, nn))
    return jnp.concatenate(outs, axis=0)


# speedup vs baseline: 1.1472x; 1.0085x over previous
import functools

import jax
import jax.numpy as jnp
from jax import lax
from jax.experimental import pallas as pl
from jax.experimental.pallas import tpu as pltpu
from jax.experimental.pallas import tpu_sc as plsc

N, K, C = 10000, 32, 128
NK = N * K

_NC, _NS = 2, 16
_NW = _NC * _NS
_NBUF = 5

_BLK = 400

_SLABS = ((0, 4800), (4800, 5200))

_INV_SQRT2 = 0.7071067811865476


def _gelu_exact(x):
    return 0.5 * x * (1.0 + lax.erf(x * _INV_SQRT2))


def _pre_body(feat_ref, w1_ref, b1_ref, a_ref, bv_ref):
    f = feat_ref[...]
    w1a = w1_ref[:C, :]
    w1b = w1_ref[C:, :]
    a_ref[...] = jnp.dot(f, w1b, preferred_element_type=jnp.float32,
                         precision=lax.Precision.HIGHEST)
    bv_ref[...] = jnp.dot(f, w1a - w1b, preferred_element_type=jnp.float32,
                          precision=lax.Precision.HIGHEST) + b1_ref[...]


def _pre_pass(feat, W1, b1):
    return pl.pallas_call(
        _pre_body,
        grid=(N // _BLK,),
        in_specs=[
            pl.BlockSpec((_BLK, C), lambda i: (i, 0)),
            pl.BlockSpec((2 * C, C), lambda i: (0, 0)),
            pl.BlockSpec((1, C), lambda i: (0, 0)),
        ],
        out_specs=[
            pl.BlockSpec((_BLK, C), lambda i: (i, 0)),
            pl.BlockSpec((_BLK, C), lambda i: (i, 0)),
        ],
        out_shape=[
            jax.ShapeDtypeStruct((N, C), jnp.float32),
            jax.ShapeDtypeStruct((N, C), jnp.float32),
        ],
    )(feat, W1, b1.reshape(1, C))


def _sc_gather_body(nch, ch, a_hbm, idx_hbm, g_hbm, idx_v, r0, r1, r2, r3, r4,
                    si0, si1, si2, si3, si4, so0, so1, so2, so3, so4):
    rows = (r0, r1, r2, r3, r4)
    sin = (si0, si1, si2, si3, si4)
    sout = (so0, so1, so2, so3, so4)
    epw = nch * ch
    wid = lax.axis_index("s") * _NC + lax.axis_index("c")
    pltpu.sync_copy(idx_hbm.at[wid], idx_v)
    base = wid * epw

    pltpu.async_copy(a_hbm.at[idx_v.at[0]], rows[0], sin[0])
    pltpu.async_copy(a_hbm.at[idx_v.at[1]], rows[1], sin[1])

    def step(j, carry):
        c0 = j * _NBUF
        for b in range(_NBUF):
            c = c0 + b
            pltpu.make_async_copy(a_hbm.at[idx_v.at[c]], rows[b],
                                  sin[b]).wait()
            pltpu.async_copy(rows[b], g_hbm.at[pl.ds(base + c * ch, ch)],
                             sout[b])
            nb = (b + 2) % _NBUF
            nc = c + 2

            @pl.when(nc >= _NBUF)
            def _():
                pltpu.make_async_copy(
                    rows[nb], g_hbm.at[pl.ds(base + (c - 3) * ch, ch)],
                    sout[nb]).wait()

            @pl.when(nc < nch)
            def _():
                pltpu.async_copy(a_hbm.at[idx_v.at[nc]], rows[nb], sin[nb])
        return carry

    lax.fori_loop(0, nch // _NBUF, step, 0)
    for c in range(nch - (_NBUF - 2), nch):
        b = c % _NBUF
        pltpu.make_async_copy(rows[b], g_hbm.at[pl.ds(base + c * ch, ch)],
                              sout[b]).wait()


@functools.cache
def _sc_gather_call(nch, ch):
    mesh = plsc.VectorSubcoreMesh(core_axis_name="c", subcore_axis_name="s")
    return pl.kernel(
        functools.partial(_sc_gather_body, nch, ch),
        out_type=jax.ShapeDtypeStruct((_NW * nch * ch, C), jnp.float32),
        mesh=mesh,
        scratch_types=(
            [pltpu.VMEM((nch, ch), jnp.int32)]
            + [pltpu.VMEM((ch, C), jnp.float32)] * _NBUF
            + [pltpu.SemaphoreType.DMA] * (2 * _NBUF)),
    )


def _sc_gather(A, idx_slab, n_nodes):
    ch = 80
    nch = n_nodes // ch
    assert n_nodes % ch == 0 and nch % _NBUF == 0
    idx = idx_slab.reshape(_NW, nch, ch)
    return _sc_gather_call(nch, ch)(A, idx)


def _main_body(g_ref, bv_ref, feat_ref, w2_ref, b2_ref, gamma_ref, beta_ref,
               o_ref):
    bv = bv_ref[...]
    w2 = w2_ref[...]
    acc = None
    for k in range(K):
        h = _gelu_exact(bv + g_ref[k])
        hk = jnp.dot(h, w2, preferred_element_type=jnp.float32,
                     precision=lax.Precision.DEFAULT)
        acc = hk if acc is None else jnp.maximum(acc, hk)
    x = acc + b2_ref[...] + feat_ref[...]
    mean = jnp.mean(x, axis=1, keepdims=True)
    var = jnp.mean((x - mean) ** 2, axis=1, keepdims=True)
    o_ref[...] = ((x - mean) * lax.rsqrt(var + 1e-5)) * gamma_ref[...] \
        + beta_ref[...]


def _main_pass(G, Bv, feat, W2, b2, gamma, beta, n0, nn):
    blk0 = n0 // _BLK
    return pl.pallas_call(
        _main_body,
        grid=(nn // _BLK,),
        in_specs=[
            pl.BlockSpec((K, _BLK, C), lambda i: (0, i, 0)),
            pl.BlockSpec((_BLK, C), lambda i: (i + blk0, 0)),
            pl.BlockSpec((_BLK, C), lambda i: (i + blk0, 0)),
            pl.BlockSpec((C, C), lambda i: (0, 0)),
            pl.BlockSpec((1, C), lambda i: (0, 0)),
            pl.BlockSpec((1, C), lambda i: (0, 0)),
            pl.BlockSpec((1, C), lambda i: (0, 0)),
        ],
        out_specs=pl.BlockSpec((_BLK, C), lambda i: (i, 0)),
        out_shape=jax.ShapeDtypeStruct((nn, C), jnp.float32),
    )(G, Bv, feat, W2, b2.reshape(1, C), gamma.reshape(1, C),
      beta.reshape(1, C))


def kernel(feat, knn_idx, W1, b1, W2, b2, gamma, beta):
    idx_t = knn_idx.astype(jnp.int32).reshape(N, K).T
    A, Bv = _pre_pass(feat, W1, b1)
    outs = []
    for n0, nn in _SLABS:
        G = _sc_gather(A, idx_t[:, n0:n0 + nn].reshape(-1), nn)
        outs.append(_main_pass(G.reshape(K, nn, C), Bv, feat, W2, b2,
                               gamma, beta, n0, nn))
    return jnp.concatenate(outs, axis=0)


# speedup vs baseline: 1.1630x; 1.0138x over previous
import functools

import jax
import jax.numpy as jnp
from jax import lax
from jax.experimental import pallas as pl
from jax.experimental.pallas import tpu as pltpu
from jax.experimental.pallas import tpu_sc as plsc

N, K, C = 10000, 32, 128
NK = N * K

_NC, _NS = 2, 16
_NW = _NC * _NS
_NBUF = 5

_BLK = 400

_SLABS = ((0, 4800), (4800, 5200))

_INV_SQRT2 = 0.7071067811865476


def _gelu_exact(x):
    return 0.5 * x * (1.0 + lax.erf(x * _INV_SQRT2))


def _pre_body(feat_ref, w1_ref, b1_ref, a_ref, bv_ref):
    f = feat_ref[...]
    w1a = w1_ref[:C, :]
    w1b = w1_ref[C:, :]
    a_ref[...] = jnp.dot(f, w1b, preferred_element_type=jnp.float32,
                         precision=lax.Precision.HIGHEST)
    bv_ref[...] = jnp.dot(f, w1a - w1b, preferred_element_type=jnp.float32,
                          precision=lax.Precision.HIGHEST) + b1_ref[...]


def _pre_pass(feat, W1, b1):
    return pl.pallas_call(
        _pre_body,
        grid=(N // _BLK,),
        in_specs=[
            pl.BlockSpec((_BLK, C), lambda i: (i, 0)),
            pl.BlockSpec((2 * C, C), lambda i: (0, 0)),
            pl.BlockSpec((1, C), lambda i: (0, 0)),
        ],
        out_specs=[
            pl.BlockSpec((_BLK, C), lambda i: (i, 0)),
            pl.BlockSpec((_BLK, C), lambda i: (i, 0)),
        ],
        out_shape=[
            jax.ShapeDtypeStruct((N, C), jnp.float32),
            jax.ShapeDtypeStruct((N, C), jnp.float32),
        ],
    )(feat, W1, b1.reshape(1, C))


def _sc_gather_body(nch, ch, a_hbm, idx_hbm, g_hbm, idx_v, r0, r1, r2, r3, r4,
                    si0, si1, si2, si3, si4, so0, so1, so2, so3, so4):
    rows = (r0, r1, r2, r3, r4)
    sin = (si0, si1, si2, si3, si4)
    sout = (so0, so1, so2, so3, so4)
    epw = nch * ch
    wid = lax.axis_index("s") * _NC + lax.axis_index("c")
    pltpu.sync_copy(idx_hbm.at[wid], idx_v)
    base = wid * epw

    pltpu.async_copy(a_hbm.at[idx_v.at[0]], rows[0], sin[0])
    pltpu.async_copy(a_hbm.at[idx_v.at[1]], rows[1], sin[1])
    pltpu.async_copy(a_hbm.at[idx_v.at[2]], rows[2], sin[2])

    def step(j, carry):
        c0 = j * _NBUF
        for b in range(_NBUF):
            c = c0 + b
            pltpu.make_async_copy(a_hbm.at[idx_v.at[c]], rows[b],
                                  sin[b]).wait()
            pltpu.async_copy(rows[b], g_hbm.at[pl.ds(base + c * ch, ch)],
                             sout[b])
            nb = (b + 3) % _NBUF
            nc = c + 3

            @pl.when(nc >= _NBUF)
            def _():
                pltpu.make_async_copy(
                    rows[nb], g_hbm.at[pl.ds(base + (c - 2) * ch, ch)],
                    sout[nb]).wait()

            @pl.when(nc < nch)
            def _():
                pltpu.async_copy(a_hbm.at[idx_v.at[nc]], rows[nb], sin[nb])
        return carry

    lax.fori_loop(0, nch // _NBUF, step, 0)
    for c in range(nch - 2, nch):
        b = c % _NBUF
        pltpu.make_async_copy(rows[b], g_hbm.at[pl.ds(base + c * ch, ch)],
                              sout[b]).wait()


@functools.cache
def _sc_gather_call(nch, ch):
    mesh = plsc.VectorSubcoreMesh(core_axis_name="c", subcore_axis_name="s")
    return pl.kernel(
        functools.partial(_sc_gather_body, nch, ch),
        out_type=jax.ShapeDtypeStruct((_NW * nch * ch, C), jnp.float32),
        mesh=mesh,
        scratch_types=(
            [pltpu.VMEM((nch, ch), jnp.int32)]
            + [pltpu.VMEM((ch, C), jnp.float32)] * _NBUF
            + [pltpu.SemaphoreType.DMA] * (2 * _NBUF)),
    )


def _sc_gather(A, idx_slab, n_nodes):
    ch = 80
    nch = n_nodes // ch
    assert n_nodes % ch == 0 and nch % _NBUF == 0
    idx = idx_slab.reshape(_NW, nch, ch)
    return _sc_gather_call(nch, ch)(A, idx)


def _main_body(g_ref, bv_ref, feat_ref, w2_ref, b2_ref, gamma_ref, beta_ref,
               o_ref):
    bv = bv_ref[...]
    w2 = w2_ref[...]
    acc = None
    for k in range(K):
        h = _gelu_exact(bv + g_ref[k])
        hk = jnp.dot(h, w2, preferred_element_type=jnp.float32,
                     precision=lax.Precision.DEFAULT)
        acc = hk if acc is None else jnp.maximum(acc, hk)
    x = acc + b2_ref[...] + feat_ref[...]
    mean = jnp.mean(x, axis=1, keepdims=True)
    var = jnp.mean((x - mean) ** 2, axis=1, keepdims=True)
    o_ref[...] = ((x - mean) * lax.rsqrt(var + 1e-5)) * gamma_ref[...] \
        + beta_ref[...]


def _main_pass(G, Bv, feat, W2, b2, gamma, beta, n0, nn):
    blk0 = n0 // _BLK
    return pl.pallas_call(
        _main_body,
        grid=(nn // _BLK,),
        in_specs=[
            pl.BlockSpec((K, _BLK, C), lambda i: (0, i, 0)),
            pl.BlockSpec((_BLK, C), lambda i: (i + blk0, 0)),
            pl.BlockSpec((_BLK, C), lambda i: (i + blk0, 0)),
            pl.BlockSpec((C, C), lambda i: (0, 0)),
            pl.BlockSpec((1, C), lambda i: (0, 0)),
            pl.BlockSpec((1, C), lambda i: (0, 0)),
            pl.BlockSpec((1, C), lambda i: (0, 0)),
        ],
        out_specs=pl.BlockSpec((_BLK, C), lambda i: (i, 0)),
        out_shape=jax.ShapeDtypeStruct((nn, C), jnp.float32),
    )(G, Bv, feat, W2, b2.reshape(1, C), gamma.reshape(1, C),
      beta.reshape(1, C))


def kernel(feat, knn_idx, W1, b1, W2, b2, gamma, beta):
    idx_t = knn_idx.astype(jnp.int32).reshape(N, K).T
    A, Bv = _pre_pass(feat, W1, b1)
    outs = []
    for n0, nn in _SLABS:
        G = _sc_gather(A, idx_t[:, n0:n0 + nn].reshape(-1), nn)
        outs.append(_main_pass(G.reshape(K, nn, C), Bv, feat, W2, b2,
                               gamma, beta, n0, nn))
    return jnp.concatenate(outs, axis=0)
